# baseline (speedup 1.0000x reference)
import jax
import jax.numpy as jnp
from jax import lax
from jax.experimental import pallas as pl
from jax.experimental.pallas import tpu as pltpu
from jax.experimental.pallas import tpu_sc as plsc

GRID_H, GRID_W, EMBED_DIM = 32, 32, 768
D = EMBED_DIM // 3
N = GRID_H * GRID_W

NC, NS = 2, 16
NW = NC * NS
L = 16


def _sc_body(row_hbm, col_hbm, time_hbm, cls_hbm, out_hbm,
             row_v, col_v, time_v, cls_v, idx_v, lsem, ssem):
    wid = lax.axis_index("s") * NC + lax.axis_index("c")
    base = 1 + GRID_W * wid

    widv = jnp.full((L,), wid, jnp.int32)
    idx_v[pl.ds(0, L)] = widv
    idx_v[pl.ds(L, L)] = widv

    c_row = pltpu.async_copy(row_hbm.at[idx_v], row_v, lsem)
    c_col = pltpu.async_copy(col_hbm, col_v, lsem)
    c_time = pltpu.async_copy(time_hbm.at[pl.ds(GRID_W * wid, GRID_W)],
                              time_v, lsem)
    c_row.wait()
    c_col.wait()
    c_time.wait()

    s_row = pltpu.async_copy(
        row_v, out_hbm.at[pl.ds(base, GRID_W), 0, pl.ds(0, D)], ssem)
    s_col = pltpu.async_copy(
        col_v, out_hbm.at[pl.ds(base, GRID_W), 0, pl.ds(D, D)], ssem)
    s_time = pltpu.async_copy(
        time_v, out_hbm.at[pl.ds(base, GRID_W), 0, pl.ds(2 * D, D)], ssem)

    @pl.when(wid == 0)
    def _():
        pltpu.sync_copy(cls_hbm, cls_v)
        pltpu.sync_copy(cls_v, out_hbm.at[pl.ds(0, 1), 0])

    s_row.wait()
    s_col.wait()
    s_time.wait()


def kernel(x, row_embed, col_embed, time_embed, cls_token_pos):
    mesh = plsc.VectorSubcoreMesh(core_axis_name="c", subcore_axis_name="s")
    cls2d = cls_token_pos.reshape(1, EMBED_DIM)
    run = pl.kernel(
        _sc_body,
        mesh=mesh,
        out_type=jax.ShapeDtypeStruct((N + 1, 1, EMBED_DIM), jnp.float32),
        scratch_types=[
            pltpu.VMEM((GRID_W, D), jnp.float32),
            pltpu.VMEM((GRID_W, D), jnp.float32),
            pltpu.VMEM((GRID_W, D), jnp.float32),
            pltpu.VMEM((1, EMBED_DIM), jnp.float32),
            pltpu.VMEM((GRID_W,), jnp.int32),
            pltpu.SemaphoreType.DMA,
            pltpu.SemaphoreType.DMA,
        ],
        compiler_params=pltpu.CompilerParams(use_tc_tiling_on_sc=False),
    )
    out = run(row_embed, col_embed, time_embed, cls2d)
    return out.reshape(1, N + 1, EMBED_DIM)


# speedup vs baseline: 8.7641x; 8.7641x over previous
import jax
import jax.numpy as jnp
from jax import lax
from jax.experimental import pallas as pl

GRID_H, GRID_W, EMBED_DIM = 32, 32, 768
D = EMBED_DIM // 3
N = GRID_H * GRID_W
BLK = 256
NBLK = (N + 1 + BLK - 1) // BLK


def _pos_emb_kernel(row_ref, col_ref, tmain_ref, tprev_ref, cls_ref, out_ref):
    b = pl.program_id(0)

    row8 = row_ref[pl.ds(pl.multiple_of(jnp.minimum(8 * b, GRID_H - 8), 8), 8)]
    rowu = jnp.broadcast_to(row8[:, None, :], (8, GRID_W, D)).reshape(BLK, D)
    colu = jnp.broadcast_to(col_ref[...][None, :, :],
                            (8, GRID_W, D)).reshape(BLK, D)
    timeu = tmain_ref[...]

    pr = row_ref[pl.ds(pl.multiple_of(jnp.maximum(8 * b - 8, 0), 8), 8)][7:8]
    pt = tprev_ref[7:8]
    pc = col_ref[GRID_W - 1:GRID_W]

    rowp = jnp.concatenate([pr, rowu[:BLK - 1]], axis=0)
    colp = jnp.concatenate([pc, colu[:BLK - 1]], axis=0)
    timep = jnp.concatenate([pt, timeu[:BLK - 1]], axis=0)

    v = jnp.concatenate([rowp, colp, timep], axis=-1)

    rid = lax.broadcasted_iota(jnp.int32, (BLK, EMBED_DIM), 0)
    clsv = jnp.broadcast_to(cls_ref[0], (BLK, EMBED_DIM))
    v = jnp.where((rid == 0) & (b == 0), clsv, v)

    out_ref[...] = v.reshape(BLK, 1, EMBED_DIM)


def kernel(x, row_embed, col_embed, time_embed, cls_token_pos):
    out = pl.pallas_call(
        _pos_emb_kernel,
        grid=(NBLK,),
        in_specs=[
            pl.BlockSpec((GRID_H, D), lambda b: (0, 0)),
            pl.BlockSpec((GRID_W, D), lambda b: (0, 0)),
            pl.BlockSpec((BLK, D), lambda b: (jnp.minimum(b, N // BLK - 1), 0)),
            pl.BlockSpec((8, D), lambda b: (jnp.maximum(32 * b - 1, 0), 0)),
            pl.BlockSpec((1, 1, EMBED_DIM), lambda b: (0, 0, 0)),
        ],
        out_specs=pl.BlockSpec((BLK, 1, EMBED_DIM), lambda b: (b, 0, 0)),
        out_shape=jax.ShapeDtypeStruct((N + 1, 1, EMBED_DIM), jnp.float32),
    )(row_embed, col_embed, time_embed, time_embed, cls_token_pos)
    return out.reshape(1, N + 1, EMBED_DIM)


# speedup vs baseline: 9.6665x; 1.1030x over previous
import jax
import jax.numpy as jnp
from jax.experimental import pallas as pl

GRID_H, GRID_W, EMBED_DIM = 32, 32, 768
D = EMBED_DIM // 3
N = GRID_H * GRID_W


def _pos_emb_kernel(row_ref, col_ref, time_ref, cls_ref, out_ref):
    row_grid = jnp.broadcast_to(row_ref[...][:, None, :], (GRID_H, GRID_W, D))
    col_grid = jnp.broadcast_to(col_ref[...][None, :, :], (GRID_H, GRID_W, D))
    out_ref[pl.ds(1, N), 0, 0:D] = row_grid.reshape(N, D)
    out_ref[pl.ds(1, N), 0, D:2 * D] = col_grid.reshape(N, D)
    out_ref[pl.ds(1, N), 0, 2 * D:3 * D] = time_ref[...]
    out_ref[0:1, 0, :] = cls_ref[0]


def kernel(x, row_embed, col_embed, time_embed, cls_token_pos):
    out = pl.pallas_call(
        _pos_emb_kernel,
        out_shape=jax.ShapeDtypeStruct((N + 1, 1, EMBED_DIM), jnp.float32),
    )(row_embed, col_embed, time_embed, cls_token_pos)
    return out.reshape(1, N + 1, EMBED_DIM)
